# Initial kernel scaffold; baseline (speedup 1.0000x reference)
import jax
import jax.numpy as jnp
from jax.experimental import pallas as pl


def kernel(x, batch):
    raise NotImplementedError("write your pallas kernel here")


# speedup vs baseline: 4.2256x; 4.2256x over previous
import functools

import jax
import jax.numpy as jnp
from jax import lax
from jax.experimental import pallas as pl
from jax.experimental.pallas import tpu as pltpu
from jax.experimental.pallas import tpu_sc as plsc

NUM_ROWS = 100000
NUM_COLS = 128
NUM_SEG = 512
NC = 2
NS = 16
ROWS_PER_CORE = NUM_ROWS // NC
BLK = 120
FULL_BLKS = ROWS_PER_CORE // BLK
BLKS_PER_TILE = FULL_BLKS // NS
REM = ROWS_PER_CORE - FULL_BLKS * BLK
SEG_PER_TILE = NUM_SEG // NS


def _seg_sum_body(x_hbm, ids_hbm, part_hbm, idx_v, idx_r, xbuf, zbuf, acc_sh):
    c = lax.axis_index("c")
    s = lax.axis_index("s")

    def zrow(i, carry):
        for j in range(NUM_COLS // 16):
            zbuf[i, pl.ds(j * 16, 16)] = jnp.zeros((16,), jnp.float32)
        return carry

    lax.fori_loop(0, SEG_PER_TILE, zrow, 0)
    pltpu.sync_copy(zbuf, acc_sh.at[pl.ds(s * SEG_PER_TILE, SEG_PER_TILE)])
    plsc.subcore_barrier()

    base = c * ROWS_PER_CORE + s * (BLKS_PER_TILE * BLK)

    def body(b, carry):
        row0 = pl.multiple_of(base + b * BLK, 8)
        pltpu.sync_copy(ids_hbm.at[pl.ds(row0, BLK)], idx_v)
        pltpu.sync_copy(x_hbm.at[pl.ds(row0, BLK)], xbuf)
        pltpu.sync_copy(xbuf, acc_sh.at[idx_v], add=True)
        return carry

    lax.fori_loop(0, BLKS_PER_TILE, body, 0)

    @pl.when(s == NS - 1)
    def _():
        row0 = pl.multiple_of(c * ROWS_PER_CORE + FULL_BLKS * BLK, 8)
        pltpu.sync_copy(ids_hbm.at[pl.ds(row0, REM)], idx_r)
        pltpu.sync_copy(x_hbm.at[pl.ds(row0, REM)], xbuf.at[pl.ds(0, REM)])
        pltpu.sync_copy(xbuf.at[pl.ds(0, REM)], acc_sh.at[idx_r], add=True)

    plsc.subcore_barrier()

    @pl.when(s == 0)
    def _():
        pltpu.sync_copy(acc_sh, part_hbm.at[c])


def _combine_body(p_ref, o_ref):
    o_ref[...] = p_ref[0] + p_ref[1]


@jax.jit
def _seg_sum(x, ids):
    mesh = plsc.VectorSubcoreMesh(core_axis_name="c", subcore_axis_name="s")
    parts = functools.partial(
        pl.kernel,
        out_type=jax.ShapeDtypeStruct((NC, NUM_SEG, NUM_COLS), jnp.float32),
        mesh=mesh,
        scratch_types=[
            pltpu.VMEM((BLK,), jnp.int32),
            pltpu.VMEM((REM,), jnp.int32),
            pltpu.VMEM((BLK, NUM_COLS), jnp.float32),
            pltpu.VMEM((SEG_PER_TILE, NUM_COLS), jnp.float32),
            pltpu.VMEM_SHARED((NUM_SEG, NUM_COLS), jnp.float32),
        ],
    )(_seg_sum_body)(x, ids)
    return pl.pallas_call(
        _combine_body,
        out_shape=jax.ShapeDtypeStruct((NUM_SEG, NUM_COLS), jnp.float32),
    )(parts)


def kernel(x, batch):
    return _seg_sum(x, batch.astype(jnp.int32))


# speedup vs baseline: 5.6444x; 1.3358x over previous
import functools

import jax
import jax.numpy as jnp
from jax import lax
from jax.experimental import pallas as pl
from jax.experimental.pallas import tpu as pltpu
from jax.experimental.pallas import tpu_sc as plsc

NUM_ROWS = 100000
NUM_COLS = 128
NUM_SEG = 512
NC = 2
NS = 16
ROWS_PER_CORE = NUM_ROWS // NC
BLK = 120
FULL_BLKS = ROWS_PER_CORE // BLK
BLKS_PER_TILE = FULL_BLKS // NS
REM = ROWS_PER_CORE - FULL_BLKS * BLK
SEG_PER_TILE = NUM_SEG // NS


NBUF = 2
GROUPS = BLKS_PER_TILE // NBUF


def _seg_sum_body(x_hbm, ids_hbm, part_hbm, idx0, idx1, idx_r, xb0, xb1,
                  zbuf, acc_sh, si0, si1, sx0, sx1):
    c = lax.axis_index("c")
    s = lax.axis_index("s")
    idxs, xbs, sis, sxs = (idx0, idx1), (xb0, xb1), (si0, si1), (sx0, sx1)

    def zrow(i, carry):
        for j in range(NUM_COLS // 16):
            zbuf[i, pl.ds(j * 16, 16)] = jnp.zeros((16,), jnp.float32)
        return carry

    lax.fori_loop(0, SEG_PER_TILE, zrow, 0)
    pltpu.sync_copy(zbuf, acc_sh.at[pl.ds(s * SEG_PER_TILE, SEG_PER_TILE)])
    plsc.subcore_barrier()

    base = c * ROWS_PER_CORE + s * (BLKS_PER_TILE * BLK)

    def start(b, k):
        row0 = pl.multiple_of(base + b * BLK, 8)
        pltpu.async_copy(ids_hbm.at[pl.ds(row0, BLK)], idxs[k], sis[k])
        pltpu.async_copy(x_hbm.at[pl.ds(row0, BLK)], xbs[k], sxs[k])

    def wait(k):
        pltpu.make_async_copy(ids_hbm.at[pl.ds(0, BLK)], idxs[k], sis[k]).wait()
        pltpu.make_async_copy(x_hbm.at[pl.ds(0, BLK)], xbs[k], sxs[k]).wait()

    for k in range(NBUF):
        start(k, k)

    def grp(g, carry):
        for k in range(NBUF):
            b = NBUF * g + k
            wait(k)
            pltpu.sync_copy(xbs[k], acc_sh.at[idxs[k]], add=True)

            @pl.when(g < GROUPS - 1)
            def _():
                start(b + NBUF, k)
        return carry

    lax.fori_loop(0, GROUPS, grp, 0)

    @pl.when(s == NS - 1)
    def _():
        row0 = pl.multiple_of(c * ROWS_PER_CORE + FULL_BLKS * BLK, 8)
        pltpu.sync_copy(ids_hbm.at[pl.ds(row0, REM)], idx_r)
        pltpu.sync_copy(x_hbm.at[pl.ds(row0, REM)], xb0.at[pl.ds(0, REM)])
        pltpu.sync_copy(xb0.at[pl.ds(0, REM)], acc_sh.at[idx_r], add=True)

    plsc.subcore_barrier()

    @pl.when(s == 0)
    def _():
        pltpu.sync_copy(acc_sh, part_hbm.at[c])


def _combine_body(p_ref, o_ref):
    o_ref[...] = p_ref[0] + p_ref[1]


@jax.jit
def _seg_sum(x, ids):
    mesh = plsc.VectorSubcoreMesh(core_axis_name="c", subcore_axis_name="s")
    parts = functools.partial(
        pl.kernel,
        out_type=jax.ShapeDtypeStruct((NC, NUM_SEG, NUM_COLS), jnp.float32),
        mesh=mesh,
        scratch_types=[
            pltpu.VMEM((BLK,), jnp.int32),
            pltpu.VMEM((BLK,), jnp.int32),
            pltpu.VMEM((REM,), jnp.int32),
            pltpu.VMEM((BLK, NUM_COLS), jnp.float32),
            pltpu.VMEM((BLK, NUM_COLS), jnp.float32),
            pltpu.VMEM((SEG_PER_TILE, NUM_COLS), jnp.float32),
            pltpu.VMEM_SHARED((NUM_SEG, NUM_COLS), jnp.float32),
            pltpu.SemaphoreType.DMA,
            pltpu.SemaphoreType.DMA,
            pltpu.SemaphoreType.DMA,
            pltpu.SemaphoreType.DMA,
        ],
    )(_seg_sum_body)(x, ids)
    return pl.pallas_call(
        _combine_body,
        out_shape=jax.ShapeDtypeStruct((NUM_SEG, NUM_COLS), jnp.float32),
    )(parts)


def kernel(x, batch):
    return _seg_sum(x, batch.astype(jnp.int32))


# speedup vs baseline: 5.8936x; 1.0442x over previous
import functools

import jax
import jax.numpy as jnp
from jax import lax
from jax.experimental import pallas as pl
from jax.experimental.pallas import tpu as pltpu
from jax.experimental.pallas import tpu_sc as plsc

NUM_ROWS = 100000
NUM_COLS = 128
NUM_SEG = 512
NC = 2
NS = 16
ROWS_PER_CORE = NUM_ROWS // NC
BLK = 120
FULL_BLKS = ROWS_PER_CORE // BLK
BLKS_PER_TILE = FULL_BLKS // NS
REM = ROWS_PER_CORE - FULL_BLKS * BLK
SEG_PER_TILE = NUM_SEG // NS
NWIN = 3
WBLK = 2
GROUPS = BLKS_PER_TILE // WBLK


def _seg_sum_body(x_hbm, ids_hbm, part_hbm, scratch):
    (idxs, idx_r, xbs, zbuf, acc_sh, sld, ssc) = scratch
    c = lax.axis_index("c")
    s = lax.axis_index("s")

    def zrow(i, carry):
        for j in range(NUM_COLS // 16):
            zbuf[i, pl.ds(j * 16, 16)] = jnp.zeros((16,), jnp.float32)
        return carry

    lax.fori_loop(0, SEG_PER_TILE, zrow, 0)
    pltpu.sync_copy(zbuf, acc_sh.at[pl.ds(s * SEG_PER_TILE, SEG_PER_TILE)])
    plsc.subcore_barrier()

    base = c * ROWS_PER_CORE + s * (BLKS_PER_TILE * BLK)

    def start_window(g, r):
        for k in range(WBLK):
            row0 = pl.multiple_of(base + (WBLK * g + k) * BLK, 8)
            i = WBLK * r + k
            pltpu.async_copy(ids_hbm.at[pl.ds(row0, BLK)], idxs[i], sld[r])
            pltpu.async_copy(x_hbm.at[pl.ds(row0, BLK)], xbs[i], sld[r])

    def wait_window_loads(r):
        for k in range(WBLK):
            i = WBLK * r + k
            pltpu.make_async_copy(ids_hbm.at[pl.ds(0, BLK)], idxs[i], sld[r]).wait()
            pltpu.make_async_copy(x_hbm.at[pl.ds(0, BLK)], xbs[i], sld[r]).wait()

    def drain_window_scatters(r):
        for k in range(WBLK):
            i = WBLK * r + k
            pltpu.make_async_copy(xbs[i], acc_sh.at[idxs[i]], ssc[r]).wait()

    start_window(0, 0)
    start_window(1, 1)

    def grp(g, carry):
        for r in range(NWIN):
            @pl.when(g % NWIN == r)
            def _():
                wait_window_loads(r)
                for k in range(WBLK):
                    i = WBLK * r + k
                    pltpu.async_copy(xbs[i], acc_sh.at[idxs[i]], ssc[r],
                                     add=True)
                rn = (r + 2) % NWIN

                @pl.when(jnp.logical_and(g >= 1, g + 2 <= GROUPS - 1))
                def _():
                    drain_window_scatters(rn)

                @pl.when(g + 2 <= GROUPS - 1)
                def _():
                    start_window(g + 2, rn)
        return carry

    lax.fori_loop(0, GROUPS, grp, 0)

    for g in (GROUPS - 3, GROUPS - 2, GROUPS - 1):
        drain_window_scatters(g % NWIN)

    @pl.when(s == NS - 1)
    def _():
        row0 = pl.multiple_of(c * ROWS_PER_CORE + FULL_BLKS * BLK, 8)
        pltpu.sync_copy(ids_hbm.at[pl.ds(row0, REM)], idx_r)
        pltpu.sync_copy(x_hbm.at[pl.ds(row0, REM)], xbs[0].at[pl.ds(0, REM)])
        pltpu.sync_copy(xbs[0].at[pl.ds(0, REM)], acc_sh.at[idx_r], add=True)

    plsc.subcore_barrier()

    @pl.when(s == 0)
    def _():
        pltpu.sync_copy(acc_sh, part_hbm.at[c])


def _body_wrapper(x_hbm, ids_hbm, part_hbm,
                  i0, i1, i2, i3, i4, i5, idx_r,
                  x0, x1, x2, x3, x4, x5, zbuf, acc_sh,
                  l0, l1, l2, c0, c1, c2):
    _seg_sum_body(x_hbm, ids_hbm, part_hbm,
                  ((i0, i1, i2, i3, i4, i5), idx_r,
                   (x0, x1, x2, x3, x4, x5), zbuf, acc_sh,
                   (l0, l1, l2), (c0, c1, c2)))


def _combine_body(p_ref, o_ref):
    o_ref[...] = p_ref[0] + p_ref[1]


@jax.jit
def _seg_sum(x, ids):
    mesh = plsc.VectorSubcoreMesh(core_axis_name="c", subcore_axis_name="s")
    parts = functools.partial(
        pl.kernel,
        out_type=jax.ShapeDtypeStruct((NC, NUM_SEG, NUM_COLS), jnp.float32),
        mesh=mesh,
        scratch_types=(
            [pltpu.VMEM((BLK,), jnp.int32)] * 6
            + [pltpu.VMEM((REM,), jnp.int32)]
            + [pltpu.VMEM((BLK, NUM_COLS), jnp.float32)] * 6
            + [pltpu.VMEM((SEG_PER_TILE, NUM_COLS), jnp.float32)]
            + [pltpu.VMEM_SHARED((NUM_SEG, NUM_COLS), jnp.float32)]
            + [pltpu.SemaphoreType.DMA] * 6
        ),
    )(_body_wrapper)(x, ids)
    return pl.pallas_call(
        _combine_body,
        out_shape=jax.ShapeDtypeStruct((NUM_SEG, NUM_COLS), jnp.float32),
    )(parts)


def kernel(x, batch):
    return _seg_sum(x, batch.astype(jnp.int32))
